# baseline (speedup 1.0000x reference)
import jax
import jax.numpy as jnp
from jax import lax
from jax.experimental import pallas as pl
from jax.experimental.pallas import tpu as pltpu
from jax.experimental.pallas import tpu_sc as plsc

B = 16384
D = 64
NW = 32
BPW = B // NW
L = 16


def _body(uids, iids, uemb, iemb, ubias, ibias, out,
          uidx_v, iidx_v, ue_v, ie_v, ub_v, ib_v, out_v, sem):
    wid = lax.axis_index("s") * 2 + lax.axis_index("c")
    base = wid * BPW

    pltpu.sync_copy(uids.at[pl.ds(base, BPW)], uidx_v)
    pltpu.sync_copy(iids.at[pl.ds(base, BPW)], iidx_v)

    c1 = pltpu.async_copy(uemb.at[uidx_v], ue_v, sem)
    c2 = pltpu.async_copy(iemb.at[iidx_v], ie_v, sem)
    c3 = pltpu.async_copy(ubias.at[uidx_v], ub_v, sem)
    c4 = pltpu.async_copy(ibias.at[iidx_v], ib_v, sem)
    c1.wait()
    c2.wait()
    c3.wait()
    c4.wait()

    lane = lax.iota(jnp.int32, L)

    def group(g, carry):
        accv = ub_v[pl.ds(g * L, L)] + ib_v[pl.ds(g * L, L)]
        for j in range(L):
            b = g * L + j
            p = ue_v[b, pl.ds(0, L)] * ie_v[b, pl.ds(0, L)]
            for c in range(1, D // L):
                p = p + ue_v[b, pl.ds(c * L, L)] * ie_v[b, pl.ds(c * L, L)]
            s = jnp.sum(p)
            accv = jnp.where(lane == j, accv + s, accv)
        out_v[pl.ds(g * L, L)] = accv
        return carry

    lax.fori_loop(0, BPW // L, group, 0)
    pltpu.sync_copy(out_v, out.at[pl.ds(base, BPW)])


def kernel(user_ids, item_ids, user_embed, item_embed, user_bias, item_bias):
    uids = user_ids.astype(jnp.int32)
    iids = item_ids.astype(jnp.int32)
    ub1 = user_bias.reshape(-1)
    ib1 = item_bias.reshape(-1)

    mesh = plsc.VectorSubcoreMesh(core_axis_name="c", subcore_axis_name="s")
    f = pl.kernel(
        _body,
        mesh=mesh,
        out_type=jax.ShapeDtypeStruct((B,), jnp.float32),
        compiler_params=pltpu.CompilerParams(
            needs_layout_passes=False, use_tc_tiling_on_sc=False
        ),
        scratch_types=[
            pltpu.VMEM((BPW,), jnp.int32),
            pltpu.VMEM((BPW,), jnp.int32),
            pltpu.VMEM((BPW, D), jnp.float32),
            pltpu.VMEM((BPW, D), jnp.float32),
            pltpu.VMEM((BPW,), jnp.float32),
            pltpu.VMEM((BPW,), jnp.float32),
            pltpu.VMEM((BPW,), jnp.float32),
            pltpu.SemaphoreType.DMA,
        ],
    )
    return f(uids, iids, user_embed, item_embed, ub1, ib1)
